# baseline (speedup 1.0000x reference)
import jax
import jax.numpy as jnp
from jax.experimental import pallas as pl
from jax.experimental.pallas import tpu as pltpu

NUM_GROUPS = 2
TOKENS_PER_GROUP = 4096
HIDDEN_DIM = 4096
NUM_EXPERTS = 64

BLOCK_T = 1024
TOTAL = NUM_GROUPS * TOKENS_PER_GROUP


def _router_block(x_ref, w_hbm, b_ref, probs_ref, logits_ref, zacc_ref,
                  wbuf, wsem):
    g = pl.program_id(0)
    i = pl.program_id(1)
    first = (g == 0) & (i == 0)

    @pl.when(first)
    def _fetch_w():
        pltpu.make_async_copy(w_hbm, wbuf, wsem).start()
        pltpu.make_async_copy(w_hbm, wbuf, wsem).wait()

    x = x_ref[0]
    logits = jax.lax.dot_general(
        x, wbuf[...], (((1,), (1,)), ((), ())),
        preferred_element_type=jnp.float32,
    ) + b_ref[...]
    m = jnp.max(logits, axis=-1, keepdims=True)
    e = jnp.exp(logits - m)
    s = jnp.sum(e, axis=-1, keepdims=True)
    logits_ref[0] = logits
    probs_ref[0] = e / s
    log_z = m + jnp.log(s)
    partial = jnp.sum(log_z * log_z)

    @pl.when(first)
    def _init():
        zacc_ref[0] = 0.0

    zacc_ref[0] += partial

    last = (g == NUM_GROUPS - 1) & (i == pl.num_programs(1) - 1)

    @pl.when(last)
    def _finalize():
        zacc_ref[0] *= 1.0 / TOTAL


def kernel(token_inputs, W, b, expert_capacity):
    del expert_capacity
    n_blocks = TOKENS_PER_GROUP // BLOCK_T
    b2 = b.reshape(1, NUM_EXPERTS)

    probs, logits, zloss = pl.pallas_call(
        _router_block,
        grid=(NUM_GROUPS, n_blocks),
        in_specs=[
            pl.BlockSpec((1, BLOCK_T, HIDDEN_DIM), lambda g, i: (g, i, 0)),
            pl.BlockSpec(memory_space=pltpu.MemorySpace.HBM),
            pl.BlockSpec((1, NUM_EXPERTS), lambda g, i: (0, 0)),
        ],
        out_specs=[
            pl.BlockSpec((1, BLOCK_T, NUM_EXPERTS), lambda g, i: (g, i, 0)),
            pl.BlockSpec((1, BLOCK_T, NUM_EXPERTS), lambda g, i: (g, i, 0)),
            pl.BlockSpec(memory_space=pltpu.MemorySpace.SMEM),
        ],
        out_shape=[
            jax.ShapeDtypeStruct((NUM_GROUPS, TOKENS_PER_GROUP, NUM_EXPERTS), jnp.float32),
            jax.ShapeDtypeStruct((NUM_GROUPS, TOKENS_PER_GROUP, NUM_EXPERTS), jnp.float32),
            jax.ShapeDtypeStruct((1,), jnp.float32),
        ],
        scratch_shapes=[
            pltpu.VMEM((NUM_EXPERTS, HIDDEN_DIM), jnp.float32),
            pltpu.SemaphoreType.DMA,
        ],
    )(token_inputs, W, b2)

    return (probs, logits, zloss[0])


# speedup vs baseline: 1.0604x; 1.0604x over previous
import jax
import jax.numpy as jnp
from jax.experimental import pallas as pl
from jax.experimental.pallas import tpu as pltpu

NUM_GROUPS = 2
TOKENS_PER_GROUP = 4096
HIDDEN_DIM = 4096
NUM_EXPERTS = 64

BLOCK_T = 1024
TOTAL = NUM_GROUPS * TOKENS_PER_GROUP


def _router_block(x_ref, w_ref, b_ref, probs_ref, logits_ref, zacc_ref):
    g = pl.program_id(0)
    i = pl.program_id(1)
    x = x_ref[0]
    w = w_ref[...]
    logits = jax.lax.dot_general(
        x, w, (((1,), (1,)), ((), ())), preferred_element_type=jnp.float32
    ) + b_ref[...]
    m = jnp.max(logits, axis=-1, keepdims=True)
    e = jnp.exp(logits - m)
    s = jnp.sum(e, axis=-1, keepdims=True)
    logits_ref[0] = logits
    probs_ref[0] = e / s
    log_z = m + jnp.log(s)
    partial = jnp.sum(log_z * log_z)

    @pl.when((g == 0) & (i == 0))
    def _init():
        zacc_ref[0] = 0.0

    zacc_ref[0] += partial

    last = (g == NUM_GROUPS - 1) & (i == pl.num_programs(1) - 1)

    @pl.when(last)
    def _finalize():
        zacc_ref[0] *= 1.0 / TOTAL


def kernel(token_inputs, W, b, expert_capacity):
    del expert_capacity
    n_blocks = TOKENS_PER_GROUP // BLOCK_T
    b2 = b.reshape(1, NUM_EXPERTS)

    probs, logits, zloss = pl.pallas_call(
        _router_block,
        grid=(NUM_GROUPS, n_blocks),
        in_specs=[
            pl.BlockSpec((1, BLOCK_T, HIDDEN_DIM), lambda g, i: (g, i, 0)),
            pl.BlockSpec((NUM_EXPERTS, HIDDEN_DIM), lambda g, i: (0, 0)),
            pl.BlockSpec((1, NUM_EXPERTS), lambda g, i: (0, 0)),
        ],
        out_specs=[
            pl.BlockSpec((1, BLOCK_T, NUM_EXPERTS), lambda g, i: (g, i, 0)),
            pl.BlockSpec((1, BLOCK_T, NUM_EXPERTS), lambda g, i: (g, i, 0)),
            pl.BlockSpec(memory_space=pltpu.MemorySpace.SMEM),
        ],
        out_shape=[
            jax.ShapeDtypeStruct((NUM_GROUPS, TOKENS_PER_GROUP, NUM_EXPERTS), jnp.float32),
            jax.ShapeDtypeStruct((NUM_GROUPS, TOKENS_PER_GROUP, NUM_EXPERTS), jnp.float32),
            jax.ShapeDtypeStruct((1,), jnp.float32),
        ],
    )(token_inputs, W, b2)

    return (probs, logits, zloss[0])
